# baseline (speedup 1.0000x reference)
import functools

import jax
import jax.numpy as jnp
from jax import lax
from jax.experimental import pallas as pl
from jax.experimental.pallas import tpu as pltpu
from jax.experimental.pallas import tpu_sc as plsc

TILE = 128
TOK = 256


def _router_body(x_ref, rw_ref, eid_ref, cnt_ref, cnt_s):
    i = pl.program_id(0)
    nsteps = pl.num_programs(0)
    E = rw_ref.shape[0]

    @pl.when(i == 0)
    def _():
        cnt_s[...] = jnp.zeros_like(cnt_s)

    logits = lax.dot_general(
        x_ref[...], rw_ref[...], (((1,), (1,)), ((), ())),
        preferred_element_type=jnp.float32)
    maxv = jnp.max(logits, axis=1, keepdims=True)
    ids = lax.broadcasted_iota(jnp.int32, logits.shape, 1)
    cand = jnp.where(logits == maxv, ids, jnp.int32(2**31 - 1))
    eid = jnp.min(cand, axis=1, keepdims=True)
    eid_ref[...] = eid
    onehot = (ids == eid).astype(jnp.int32)
    cnt_s[0:1, 0:E] += jnp.sum(onehot, axis=0, keepdims=True)

    @pl.when(i == nsteps - 1)
    def _():
        cnt_ref[...] = cnt_s[0:1, 0:E]


def _route(x, router_w):
    S, H = x.shape
    E = router_w.shape[0]
    eids, counts = pl.pallas_call(
        _router_body,
        grid=(S // TOK,),
        in_specs=[
            pl.BlockSpec((TOK, H), lambda i: (i, 0)),
            pl.BlockSpec((E, H), lambda i: (0, 0)),
        ],
        out_specs=[
            pl.BlockSpec((TOK, 1), lambda i: (i, 0)),
            pl.BlockSpec((1, E), lambda i: (0, 0)),
        ],
        out_shape=[
            jax.ShapeDtypeStruct((S, 1), jnp.int32),
            jax.ShapeDtypeStruct((1, E), jnp.int32),
        ],
        scratch_shapes=[pltpu.VMEM((8, 128), jnp.int32)],
    )(x, router_w)
    return eids[:, 0], counts


def _col(row_1n):
    n = row_1n.shape[1]
    eye = (lax.broadcasted_iota(jnp.int32, (n, n), 0)
           == lax.broadcasted_iota(jnp.int32, (n, n), 1)).astype(jnp.float32)
    return lax.dot_general(eye, row_1n.astype(jnp.float32),
                           (((1,), (1,)), ((), ())),
                           preferred_element_type=jnp.float32)


def _meta_body(S, E, cnt_ref, meta_ref):
    NT = S // TILE
    G = E + NT - 1
    counts = cnt_ref[...].astype(jnp.float32)
    lt = (lax.broadcasted_iota(jnp.int32, (E, E), 0)
          < lax.broadcasted_iota(jnp.int32, (E, E), 1)).astype(jnp.float32)
    offsets = lax.dot_general(counts, lt, (((1,), (0,)), ((), ())),
                              preferred_element_type=jnp.float32)
    bounds = ((lax.broadcasted_iota(jnp.int32, (1, NT - 1), 1) + 1)
              * TILE).astype(jnp.float32)
    c = jnp.concatenate([offsets, bounds], axis=1)
    c_col = _col(c)
    krow = lax.broadcasted_iota(jnp.int32, (G, G), 1)
    kcol = lax.broadcasted_iota(jnp.int32, (G, G), 0)
    gt = (c_col > c).astype(jnp.int32)
    tie = ((c_col == c) & (krow < kcol)).astype(jnp.int32)
    p = jnp.sum(gt + tie, axis=1, keepdims=True)
    grow = lax.broadcasted_iota(jnp.int32, (G, G), 1)
    onehot_p = (p == grow).astype(jnp.float32)
    starts = jnp.sum(onehot_p * c_col, axis=0, keepdims=True)
    ends = jnp.concatenate(
        [starts[:, 1:], jnp.full((1, 1), float(S), jnp.float32)], axis=1)
    off_col = _col(offsets)
    seg_e = jnp.sum((off_col <= starts).astype(jnp.int32), axis=0,
                    keepdims=True) - 1
    starts_i = starts.astype(jnp.int32)
    ends_i = ends.astype(jnp.int32)
    seg_t = jnp.minimum(starts_i, S - 1) // TILE
    seg_rs = starts_i - seg_t * TILE
    seg_re = ends_i - seg_t * TILE
    prev_t = jnp.concatenate(
        [jnp.full((1, 1), -1, jnp.int32), seg_t[:, :-1]], axis=1)
    seg_ft = (seg_t != prev_t).astype(jnp.int32)

    pad = meta_ref.shape[1] - G
    z = jnp.zeros((1, pad), jnp.int32)
    rows = [jnp.concatenate([r, z], axis=1)
            for r in (seg_t, seg_e, seg_rs, seg_re, seg_ft)]
    rows += [jnp.zeros((1, meta_ref.shape[1]), jnp.int32)] * (
        meta_ref.shape[0] - len(rows))
    meta_ref[...] = jnp.concatenate(rows, axis=0)


def _meta(counts, S, E):
    return pl.pallas_call(
        functools.partial(_meta_body, S, E),
        out_shape=jax.ShapeDtypeStruct((8, 128), jnp.int32),
    )(counts)


def _sc_gather(table, idx):
    R, D = table.shape
    info = plsc.get_sparse_core_info()
    NC, NS = info.num_cores, info.num_subcores
    NW = NC * NS
    per_w = R // NW
    CH = min(per_w, 32)
    mesh = plsc.VectorSubcoreMesh(core_axis_name="c", subcore_axis_name="s")

    @functools.partial(
        pl.kernel, mesh=mesh,
        out_type=jax.ShapeDtypeStruct((R, D), table.dtype),
        scratch_types=[
            pltpu.VMEM((CH,), jnp.int32),
            pltpu.VMEM((CH, D), table.dtype),
            pltpu.SemaphoreType.DMA,
        ],
    )
    def k(tab_hbm, idx_hbm, out_hbm, idx_v, rows_v, sem):
        wid = lax.axis_index("s") * NC + lax.axis_index("c")
        for c in range(per_w // CH):
            base = wid * per_w + c * CH
            pltpu.sync_copy(idx_hbm.at[pl.ds(base, CH)], idx_v)
            pltpu.async_copy(tab_hbm.at[idx_v], rows_v, sem).wait()
            pltpu.sync_copy(rows_v, out_hbm.at[pl.ds(base, CH)])

    return k(table, idx)


def _sc_scatter(src, idx, R):
    Rs, D = src.shape
    info = plsc.get_sparse_core_info()
    NC, NS = info.num_cores, info.num_subcores
    NW = NC * NS
    per_w = Rs // NW
    CH = min(per_w, 32)
    mesh = plsc.VectorSubcoreMesh(core_axis_name="c", subcore_axis_name="s")

    @functools.partial(
        pl.kernel, mesh=mesh,
        out_type=jax.ShapeDtypeStruct((R, D), src.dtype),
        scratch_types=[
            pltpu.VMEM((CH,), jnp.int32),
            pltpu.VMEM((CH, D), src.dtype),
            pltpu.SemaphoreType.DMA,
        ],
    )
    def k(src_hbm, idx_hbm, out_hbm, idx_v, rows_v, sem):
        wid = lax.axis_index("s") * NC + lax.axis_index("c")
        for c in range(per_w // CH):
            base = wid * per_w + c * CH
            pltpu.sync_copy(idx_hbm.at[pl.ds(base, CH)], idx_v)
            pltpu.sync_copy(src_hbm.at[pl.ds(base, CH)], rows_v)
            pltpu.async_copy(rows_v, out_hbm.at[idx_v], sem).wait()

    return k(src, idx)


def _gmm_body(m_r, xs_ref, wg_ref, wu_ref, wd_ref, out_ref):
    g = pl.program_id(0)
    rs, re, ft = m_r[2, g], m_r[3, g], m_r[4, g]
    x = xs_ref[...]
    gate = jnp.dot(x, wg_ref[0], preferred_element_type=jnp.float32)
    up = jnp.dot(x, wu_ref[0], preferred_element_type=jnp.float32)
    h = gate * jax.nn.sigmoid(gate) * up
    o = jnp.dot(h, wd_ref[0], preferred_element_type=jnp.float32)
    rows = lax.broadcasted_iota(jnp.int32, o.shape, 0)
    contrib = jnp.where((rows >= rs) & (rows < re), o, 0.0)

    @pl.when(ft == 1)
    def _():
        out_ref[...] = contrib

    @pl.when(ft == 0)
    def _():
        out_ref[...] += contrib


def _gmm(meta, xs, w_gate, w_up, w_down):
    S, H = xs.shape
    E, _, I = w_gate.shape
    G = E + S // TILE - 1
    grid_spec = pltpu.PrefetchScalarGridSpec(
        num_scalar_prefetch=1,
        grid=(G,),
        in_specs=[
            pl.BlockSpec((TILE, H), lambda g, m: (m[0, g], 0)),
            pl.BlockSpec((1, H, I), lambda g, m: (m[1, g], 0, 0)),
            pl.BlockSpec((1, H, I), lambda g, m: (m[1, g], 0, 0)),
            pl.BlockSpec((1, I, H), lambda g, m: (m[1, g], 0, 0)),
        ],
        out_specs=pl.BlockSpec((TILE, H), lambda g, m: (m[0, g], 0)),
    )
    return pl.pallas_call(
        _gmm_body,
        grid_spec=grid_spec,
        out_shape=jax.ShapeDtypeStruct((S, H), jnp.float32),
        compiler_params=pltpu.CompilerParams(
            dimension_semantics=("arbitrary",)),
    )(meta, xs, w_gate, w_up, w_down)


def kernel(hidden_states, router_w, w_gate, w_up, w_down):
    B, S, H = hidden_states.shape
    E = w_gate.shape[0]
    x = hidden_states.reshape(S, H)

    eids, counts = _route(x, router_w)
    meta = _meta(counts, S, E)
    perm = jnp.argsort(eids).astype(jnp.int32)

    xs = _sc_gather(x, perm)
    outs = _gmm(meta, xs, w_gate, w_up, w_down)
    out = _sc_scatter(outs, perm, S)
    return out.reshape(B, S, H)


# speedup vs baseline: 1.0287x; 1.0287x over previous
import functools

import jax
import jax.numpy as jnp
from jax import lax
from jax.experimental import pallas as pl
from jax.experimental.pallas import tpu as pltpu
from jax.experimental.pallas import tpu_sc as plsc

TILE = 256
TOK = 256


def _router_body(x_ref, rw_ref, eid_ref, cnt_ref, cnt_s):
    i = pl.program_id(0)
    nsteps = pl.num_programs(0)
    E = rw_ref.shape[0]

    @pl.when(i == 0)
    def _():
        cnt_s[...] = jnp.zeros_like(cnt_s)

    logits = lax.dot_general(
        x_ref[...], rw_ref[...], (((1,), (1,)), ((), ())),
        preferred_element_type=jnp.float32)
    maxv = jnp.max(logits, axis=1, keepdims=True)
    ids = lax.broadcasted_iota(jnp.int32, logits.shape, 1)
    cand = jnp.where(logits == maxv, ids, jnp.int32(2**31 - 1))
    eid = jnp.min(cand, axis=1, keepdims=True)
    eid_ref[...] = eid
    onehot = (ids == eid).astype(jnp.int32)
    cnt_s[0:1, 0:E] += jnp.sum(onehot, axis=0, keepdims=True)

    @pl.when(i == nsteps - 1)
    def _():
        cnt_ref[...] = cnt_s[0:1, 0:E]


def _route(x, router_w):
    S, H = x.shape
    E = router_w.shape[0]
    eids, counts = pl.pallas_call(
        _router_body,
        grid=(S // TOK,),
        in_specs=[
            pl.BlockSpec((TOK, H), lambda i: (i, 0)),
            pl.BlockSpec((E, H), lambda i: (0, 0)),
        ],
        out_specs=[
            pl.BlockSpec((TOK, 1), lambda i: (i, 0)),
            pl.BlockSpec((1, E), lambda i: (0, 0)),
        ],
        out_shape=[
            jax.ShapeDtypeStruct((S, 1), jnp.int32),
            jax.ShapeDtypeStruct((1, E), jnp.int32),
        ],
        scratch_shapes=[pltpu.VMEM((8, 128), jnp.int32)],
    )(x, router_w)
    return eids[:, 0], counts


def _col(row_1n):
    n = row_1n.shape[1]
    eye = (lax.broadcasted_iota(jnp.int32, (n, n), 0)
           == lax.broadcasted_iota(jnp.int32, (n, n), 1)).astype(jnp.float32)
    return lax.dot_general(eye, row_1n.astype(jnp.float32),
                           (((1,), (1,)), ((), ())),
                           preferred_element_type=jnp.float32)


def _meta_body(S, E, cnt_ref, meta_ref):
    NT = S // TILE
    G = E + NT - 1
    counts = cnt_ref[...].astype(jnp.float32)
    lt = (lax.broadcasted_iota(jnp.int32, (E, E), 0)
          < lax.broadcasted_iota(jnp.int32, (E, E), 1)).astype(jnp.float32)
    offsets = lax.dot_general(counts, lt, (((1,), (0,)), ((), ())),
                              preferred_element_type=jnp.float32)
    bounds = ((lax.broadcasted_iota(jnp.int32, (1, NT - 1), 1) + 1)
              * TILE).astype(jnp.float32)
    c = jnp.concatenate([offsets, bounds], axis=1)
    c_col = _col(c)
    krow = lax.broadcasted_iota(jnp.int32, (G, G), 1)
    kcol = lax.broadcasted_iota(jnp.int32, (G, G), 0)
    gt = (c_col > c).astype(jnp.int32)
    tie = ((c_col == c) & (krow < kcol)).astype(jnp.int32)
    p = jnp.sum(gt + tie, axis=1, keepdims=True)
    grow = lax.broadcasted_iota(jnp.int32, (G, G), 1)
    onehot_p = (p == grow).astype(jnp.float32)
    starts = jnp.sum(onehot_p * c_col, axis=0, keepdims=True)
    ends = jnp.concatenate(
        [starts[:, 1:], jnp.full((1, 1), float(S), jnp.float32)], axis=1)
    off_col = _col(offsets)
    seg_e = jnp.sum((off_col <= starts).astype(jnp.int32), axis=0,
                    keepdims=True) - 1
    starts_i = starts.astype(jnp.int32)
    ends_i = ends.astype(jnp.int32)
    seg_t = jnp.minimum(starts_i, S - 1) // TILE
    seg_rs = starts_i - seg_t * TILE
    seg_re = ends_i - seg_t * TILE
    prev_t = jnp.concatenate(
        [jnp.full((1, 1), -1, jnp.int32), seg_t[:, :-1]], axis=1)
    seg_ft = (seg_t != prev_t).astype(jnp.int32)

    pad = meta_ref.shape[1] - G
    z = jnp.zeros((1, pad), jnp.int32)
    rows = [jnp.concatenate([r, z], axis=1)
            for r in (seg_t, seg_e, seg_rs, seg_re, seg_ft)]
    rows += [jnp.zeros((1, meta_ref.shape[1]), jnp.int32)] * (
        meta_ref.shape[0] - len(rows))
    meta_ref[...] = jnp.concatenate(rows, axis=0)


def _meta(counts, S, E):
    return pl.pallas_call(
        functools.partial(_meta_body, S, E),
        out_shape=jax.ShapeDtypeStruct((8, 128), jnp.int32),
    )(counts)


def _sc_gather(table, idx):
    R, D = table.shape
    info = plsc.get_sparse_core_info()
    NC, NS = info.num_cores, info.num_subcores
    NW = NC * NS
    per_w = R // NW
    CH = min(per_w, 32)
    mesh = plsc.VectorSubcoreMesh(core_axis_name="c", subcore_axis_name="s")

    @functools.partial(
        pl.kernel, mesh=mesh,
        out_type=jax.ShapeDtypeStruct((R, D), table.dtype),
        scratch_types=[
            pltpu.VMEM((CH,), jnp.int32),
            pltpu.VMEM((CH, D), table.dtype),
            pltpu.SemaphoreType.DMA,
        ],
    )
    def k(tab_hbm, idx_hbm, out_hbm, idx_v, rows_v, sem):
        wid = lax.axis_index("s") * NC + lax.axis_index("c")
        for c in range(per_w // CH):
            base = wid * per_w + c * CH
            pltpu.sync_copy(idx_hbm.at[pl.ds(base, CH)], idx_v)
            pltpu.async_copy(tab_hbm.at[idx_v], rows_v, sem).wait()
            pltpu.sync_copy(rows_v, out_hbm.at[pl.ds(base, CH)])

    return k(table, idx)


def _sc_scatter(src, idx, R):
    Rs, D = src.shape
    info = plsc.get_sparse_core_info()
    NC, NS = info.num_cores, info.num_subcores
    NW = NC * NS
    per_w = Rs // NW
    CH = min(per_w, 32)
    mesh = plsc.VectorSubcoreMesh(core_axis_name="c", subcore_axis_name="s")

    @functools.partial(
        pl.kernel, mesh=mesh,
        out_type=jax.ShapeDtypeStruct((R, D), src.dtype),
        scratch_types=[
            pltpu.VMEM((CH,), jnp.int32),
            pltpu.VMEM((CH, D), src.dtype),
            pltpu.SemaphoreType.DMA,
        ],
    )
    def k(src_hbm, idx_hbm, out_hbm, idx_v, rows_v, sem):
        wid = lax.axis_index("s") * NC + lax.axis_index("c")
        for c in range(per_w // CH):
            base = wid * per_w + c * CH
            pltpu.sync_copy(idx_hbm.at[pl.ds(base, CH)], idx_v)
            pltpu.sync_copy(src_hbm.at[pl.ds(base, CH)], rows_v)
            pltpu.async_copy(rows_v, out_hbm.at[idx_v], sem).wait()

    return k(src, idx)


def _gmm_body(m_r, xs_ref, wg_ref, wu_ref, wd_ref, out_ref):
    g = pl.program_id(0)
    rs, re, ft = m_r[2, g], m_r[3, g], m_r[4, g]
    x = xs_ref[...]
    gate = jnp.dot(x, wg_ref[0], preferred_element_type=jnp.float32)
    up = jnp.dot(x, wu_ref[0], preferred_element_type=jnp.float32)
    h = gate * jax.nn.sigmoid(gate) * up
    o = jnp.dot(h, wd_ref[0], preferred_element_type=jnp.float32)
    rows = lax.broadcasted_iota(jnp.int32, o.shape, 0)
    contrib = jnp.where((rows >= rs) & (rows < re), o, 0.0)

    @pl.when(ft == 1)
    def _():
        out_ref[...] = contrib

    @pl.when(ft == 0)
    def _():
        out_ref[...] += contrib


def _gmm(meta, xs, w_gate, w_up, w_down):
    S, H = xs.shape
    E, _, I = w_gate.shape
    G = E + S // TILE - 1
    grid_spec = pltpu.PrefetchScalarGridSpec(
        num_scalar_prefetch=1,
        grid=(G,),
        in_specs=[
            pl.BlockSpec((TILE, H), lambda g, m: (m[0, g], 0)),
            pl.BlockSpec((1, H, I), lambda g, m: (m[1, g], 0, 0)),
            pl.BlockSpec((1, H, I), lambda g, m: (m[1, g], 0, 0)),
            pl.BlockSpec((1, I, H), lambda g, m: (m[1, g], 0, 0)),
        ],
        out_specs=pl.BlockSpec((TILE, H), lambda g, m: (m[0, g], 0)),
    )
    return pl.pallas_call(
        _gmm_body,
        grid_spec=grid_spec,
        out_shape=jax.ShapeDtypeStruct((S, H), jnp.float32),
        compiler_params=pltpu.CompilerParams(
            dimension_semantics=("arbitrary",)),
    )(meta, xs, w_gate, w_up, w_down)


def kernel(hidden_states, router_w, w_gate, w_up, w_down):
    B, S, H = hidden_states.shape
    E = w_gate.shape[0]
    x = hidden_states.reshape(S, H)

    eids, counts = _route(x, router_w)
    meta = _meta(counts, S, E)
    perm = jnp.argsort(eids).astype(jnp.int32)

    xs = _sc_gather(x, perm)
    outs = _gmm(meta, xs, w_gate, w_up, w_down)
    out = _sc_scatter(outs, perm, S)
    return out.reshape(B, S, H)
